# baseline (speedup 1.0000x reference)
import jax
import jax.numpy as jnp
from jax.experimental import pallas as pl

B, S, HID = 2, 2048, 2048
H, KVH, D = 16, 4, 128
THETA = 10000.0

BM = 256
BQ = 256
BK = 256
SCALE = 1.0 / (D ** 0.5)


def _rope(x, cos, sin):
    rot = jnp.concatenate([-x[:, D // 2:], x[:, : D // 2]], axis=1)
    return x * cos + rot * sin


def _q_proj_kernel(x_ref, wq_ref, cos_ref, sin_ref, q_ref):
    x = x_ref[0]
    cos = cos_ref[...]
    sin = sin_ref[...]
    y = jnp.dot(x, wq_ref[...], preferred_element_type=jnp.float32)
    for h in range(H):
        q_ref[0, h, :, :] = _rope(y[:, h * D:(h + 1) * D], cos, sin).astype(jnp.bfloat16)


def _kv_proj_kernel(x_ref, wkv_ref, cos_ref, sin_ref, k_ref, v_ref):
    x = x_ref[0]
    cos = cos_ref[...]
    sin = sin_ref[...]
    y = jnp.dot(x, wkv_ref[...], preferred_element_type=jnp.float32)
    for h in range(KVH):
        k_ref[0, h, :, :] = _rope(y[:, h * D:(h + 1) * D], cos, sin).astype(jnp.bfloat16)
        v_ref[0, h, :, :] = y[:, (KVH + h) * D:(KVH + h + 1) * D].astype(jnp.bfloat16)


def _flash_kernel(q_ref, k_ref, v_ref, o_ref):
    qi = pl.program_id(2)
    q = q_ref[0, 0]
    rows = qi * BQ + jax.lax.broadcasted_iota(jnp.int32, (BQ, BK), 0)

    m0 = jnp.full((BQ, 1), -1e30, jnp.float32)
    l0 = jnp.zeros((BQ, 1), jnp.float32)
    acc0 = jnp.zeros((BQ, D), jnp.float32)

    def body(kb, carry):
        m, l, acc = carry
        ks = k_ref[0, 0, pl.ds(kb * BK, BK), :]
        vs = v_ref[0, 0, pl.ds(kb * BK, BK), :]
        s = jax.lax.dot_general(q, ks, (((1,), (1,)), ((), ())),
                                preferred_element_type=jnp.float32) * SCALE
        cols = kb * BK + jax.lax.broadcasted_iota(jnp.int32, (BQ, BK), 1)
        s = jnp.where(cols <= rows, s, -1e30)
        m_new = jnp.maximum(m, s.max(axis=1, keepdims=True))
        alpha = jnp.exp(m - m_new)
        p = jnp.exp(s - m_new)
        l_new = l * alpha + p.sum(axis=1, keepdims=True)
        acc_new = acc * alpha + jnp.dot(p.astype(jnp.bfloat16), vs,
                                        preferred_element_type=jnp.float32)
        return m_new, l_new, acc_new

    m, l, acc = jax.lax.fori_loop(0, qi + 1, body, (m0, l0, acc0))
    o_ref[0, 0] = (acc / l).astype(jnp.bfloat16)


def _out_proj_kernel(x_ref, wo_ref, o_ref):
    x = jnp.concatenate([x_ref[0, h] for h in range(H)], axis=1)
    o_ref[0] = jnp.dot(x, wo_ref[...], preferred_element_type=jnp.float32)


def kernel(hidden_states, Wq, Wk, Wv, Wo):
    inv_freq = 1.0 / (THETA ** (jnp.arange(0, D, 2, dtype=jnp.float32) / D))
    t = jnp.arange(S, dtype=jnp.float32)
    freqs = jnp.outer(t, inv_freq)
    emb = jnp.concatenate([freqs, freqs], axis=-1)
    cos = jnp.cos(emb)
    sin = jnp.sin(emb)
    wkv = jnp.concatenate([Wk, Wv], axis=1).astype(jnp.bfloat16)
    x16 = hidden_states.astype(jnp.bfloat16)
    wq16 = Wq.astype(jnp.bfloat16)
    wo16 = Wo.astype(jnp.bfloat16)

    q = pl.pallas_call(
        _q_proj_kernel,
        grid=(B, S // BM),
        in_specs=[
            pl.BlockSpec((1, BM, HID), lambda b, m: (b, m, 0)),
            pl.BlockSpec((HID, H * D), lambda b, m: (0, 0)),
            pl.BlockSpec((BM, D), lambda b, m: (m, 0)),
            pl.BlockSpec((BM, D), lambda b, m: (m, 0)),
        ],
        out_specs=pl.BlockSpec((1, H, BM, D), lambda b, m: (b, 0, m, 0)),
        out_shape=jax.ShapeDtypeStruct((B, H, S, D), jnp.bfloat16),
    )(x16, wq16, cos, sin)

    k, v = pl.pallas_call(
        _kv_proj_kernel,
        grid=(B, S // BM),
        in_specs=[
            pl.BlockSpec((1, BM, HID), lambda b, m: (b, m, 0)),
            pl.BlockSpec((HID, 2 * KVH * D), lambda b, m: (0, 0)),
            pl.BlockSpec((BM, D), lambda b, m: (m, 0)),
            pl.BlockSpec((BM, D), lambda b, m: (m, 0)),
        ],
        out_specs=[
            pl.BlockSpec((1, KVH, BM, D), lambda b, m: (b, 0, m, 0)),
            pl.BlockSpec((1, KVH, BM, D), lambda b, m: (b, 0, m, 0)),
        ],
        out_shape=[
            jax.ShapeDtypeStruct((B, KVH, S, D), jnp.bfloat16),
            jax.ShapeDtypeStruct((B, KVH, S, D), jnp.bfloat16),
        ],
    )(x16, wkv, cos, sin)

    o = pl.pallas_call(
        _flash_kernel,
        grid=(B, H, S // BQ),
        in_specs=[
            pl.BlockSpec((1, 1, BQ, D), lambda b, h, i: (b, h, i, 0)),
            pl.BlockSpec((1, 1, S, D), lambda b, h, i: (b, h // (H // KVH), 0, 0)),
            pl.BlockSpec((1, 1, S, D), lambda b, h, i: (b, h // (H // KVH), 0, 0)),
        ],
        out_specs=pl.BlockSpec((1, 1, BQ, D), lambda b, h, i: (b, h, i, 0)),
        out_shape=jax.ShapeDtypeStruct((B, H, S, D), jnp.bfloat16),
    )(q, k, v)

    out = pl.pallas_call(
        _out_proj_kernel,
        grid=(B, S // BM),
        in_specs=[
            pl.BlockSpec((1, H, BM, D), lambda b, m: (b, 0, m, 0)),
            pl.BlockSpec((HID, HID), lambda b, m: (0, 0)),
        ],
        out_specs=pl.BlockSpec((1, BM, HID), lambda b, m: (b, m, 0)),
        out_shape=jax.ShapeDtypeStruct((B, S, HID), jnp.float32),
    )(o, wo16)

    return out


# speedup vs baseline: 1.6567x; 1.6567x over previous
import jax
import jax.numpy as jnp
from jax.experimental import pallas as pl
from jax.experimental.pallas import tpu as pltpu

B, S, HID = 2, 2048, 2048
H, KVH, D = 16, 4, 128
THETA = 10000.0

BM = 256
BQ = 512
BK = 512
SCALE = 1.0 / (D ** 0.5)


def _rope(x, cos, sin):
    rot = jnp.concatenate([-x[:, D // 2:], x[:, : D // 2]], axis=1)
    return x * cos + rot * sin


def _q_proj_kernel(x_ref, wq_ref, cos_ref, sin_ref, q_ref):
    x = x_ref[0]
    cos = cos_ref[...]
    sin = sin_ref[...]
    y = jnp.dot(x, wq_ref[...], preferred_element_type=jnp.float32)
    for h in range(H):
        q_ref[0, h, :, :] = _rope(y[:, h * D:(h + 1) * D], cos, sin).astype(jnp.bfloat16)


def _kv_proj_kernel(x_ref, wkv_ref, cos_ref, sin_ref, k_ref, v_ref):
    x = x_ref[0]
    cos = cos_ref[...]
    sin = sin_ref[...]
    y = jnp.dot(x, wkv_ref[...], preferred_element_type=jnp.float32)
    for h in range(KVH):
        k_ref[0, h, :, :] = _rope(y[:, h * D:(h + 1) * D], cos, sin).astype(jnp.bfloat16)
        v_ref[0, h, :, :] = y[:, (KVH + h) * D:(KVH + h + 1) * D].astype(jnp.bfloat16)


def _flash_kernel(q_ref, k_ref, v_ref, o_ref):
    qi = pl.program_id(2)
    q = q_ref[0, 0]
    rows = qi * BQ + jax.lax.broadcasted_iota(jnp.int32, (BQ, BK), 0)

    m0 = jnp.full((BQ, 1), -1e30, jnp.float32)
    l0 = jnp.zeros((BQ, 1), jnp.float32)
    acc0 = jnp.zeros((BQ, D), jnp.float32)

    def body(kb, carry):
        m, l, acc = carry
        ks = k_ref[0, 0, pl.ds(kb * BK, BK), :]
        vs = v_ref[0, 0, pl.ds(kb * BK, BK), :]
        s = jax.lax.dot_general(q, ks, (((1,), (1,)), ((), ())),
                                preferred_element_type=jnp.float32) * SCALE
        cols = kb * BK + jax.lax.broadcasted_iota(jnp.int32, (BQ, BK), 1)
        s = jnp.where(cols <= rows, s, -1e30)
        m_new = jnp.maximum(m, s.max(axis=1, keepdims=True))
        alpha = jnp.exp(m - m_new)
        p = jnp.exp(s - m_new)
        l_new = l * alpha + p.sum(axis=1, keepdims=True)
        acc_new = acc * alpha + jnp.dot(p.astype(jnp.bfloat16), vs,
                                        preferred_element_type=jnp.float32)
        return m_new, l_new, acc_new

    m, l, acc = jax.lax.fori_loop(0, qi + 1, body, (m0, l0, acc0))
    o_ref[0, 0] = (acc / l).astype(jnp.bfloat16)


def _out_proj_kernel(x_ref, wo_ref, o_ref):
    x = jnp.concatenate([x_ref[0, h] for h in range(H)], axis=1)
    o_ref[0] = jnp.dot(x, wo_ref[...], preferred_element_type=jnp.float32)


def kernel(hidden_states, Wq, Wk, Wv, Wo):
    inv_freq = 1.0 / (THETA ** (jnp.arange(0, D, 2, dtype=jnp.float32) / D))
    t = jnp.arange(S, dtype=jnp.float32)
    freqs = jnp.outer(t, inv_freq)
    emb = jnp.concatenate([freqs, freqs], axis=-1)
    cos = jnp.cos(emb)
    sin = jnp.sin(emb)
    wkv = jnp.concatenate([Wk, Wv], axis=1).astype(jnp.bfloat16)
    x16 = hidden_states.astype(jnp.bfloat16)
    wq16 = Wq.astype(jnp.bfloat16)
    wo16 = Wo.astype(jnp.bfloat16)

    q = pl.pallas_call(
        _q_proj_kernel,
        grid=(B, S // BM),
        in_specs=[
            pl.BlockSpec((1, BM, HID), lambda b, m: (b, m, 0)),
            pl.BlockSpec((HID, H * D), lambda b, m: (0, 0)),
            pl.BlockSpec((BM, D), lambda b, m: (m, 0)),
            pl.BlockSpec((BM, D), lambda b, m: (m, 0)),
        ],
        out_specs=pl.BlockSpec((1, H, BM, D), lambda b, m: (b, 0, m, 0)),
        out_shape=jax.ShapeDtypeStruct((B, H, S, D), jnp.bfloat16),
        compiler_params=pltpu.CompilerParams(
            dimension_semantics=("parallel", "arbitrary")),
    )(x16, wq16, cos, sin)

    k, v = pl.pallas_call(
        _kv_proj_kernel,
        grid=(B, S // BM),
        in_specs=[
            pl.BlockSpec((1, BM, HID), lambda b, m: (b, m, 0)),
            pl.BlockSpec((HID, 2 * KVH * D), lambda b, m: (0, 0)),
            pl.BlockSpec((BM, D), lambda b, m: (m, 0)),
            pl.BlockSpec((BM, D), lambda b, m: (m, 0)),
        ],
        out_specs=[
            pl.BlockSpec((1, KVH, BM, D), lambda b, m: (b, 0, m, 0)),
            pl.BlockSpec((1, KVH, BM, D), lambda b, m: (b, 0, m, 0)),
        ],
        out_shape=[
            jax.ShapeDtypeStruct((B, KVH, S, D), jnp.bfloat16),
            jax.ShapeDtypeStruct((B, KVH, S, D), jnp.bfloat16),
        ],
        compiler_params=pltpu.CompilerParams(
            dimension_semantics=("parallel", "arbitrary")),
    )(x16, wkv, cos, sin)

    o = pl.pallas_call(
        _flash_kernel,
        grid=(B, H, S // BQ),
        in_specs=[
            pl.BlockSpec((1, 1, BQ, D), lambda b, h, i: (b, h, i, 0)),
            pl.BlockSpec((1, 1, S, D), lambda b, h, i: (b, h // (H // KVH), 0, 0)),
            pl.BlockSpec((1, 1, S, D), lambda b, h, i: (b, h // (H // KVH), 0, 0)),
        ],
        out_specs=pl.BlockSpec((1, 1, BQ, D), lambda b, h, i: (b, h, i, 0)),
        out_shape=jax.ShapeDtypeStruct((B, H, S, D), jnp.bfloat16),
        compiler_params=pltpu.CompilerParams(
            dimension_semantics=("parallel", "parallel", "arbitrary")),
    )(q, k, v)

    out = pl.pallas_call(
        _out_proj_kernel,
        grid=(B, S // BM),
        in_specs=[
            pl.BlockSpec((1, H, BM, D), lambda b, m: (b, 0, m, 0)),
            pl.BlockSpec((HID, HID), lambda b, m: (0, 0)),
        ],
        out_specs=pl.BlockSpec((1, BM, HID), lambda b, m: (b, m, 0)),
        out_shape=jax.ShapeDtypeStruct((B, S, HID), jnp.float32),
        compiler_params=pltpu.CompilerParams(
            dimension_semantics=("parallel", "arbitrary")),
    )(o, wo16)

    return out


# speedup vs baseline: 2.0537x; 1.2396x over previous
import jax
import jax.numpy as jnp
from jax.experimental import pallas as pl
from jax.experimental.pallas import tpu as pltpu

B, S, HID = 2, 2048, 2048
H, KVH, D = 16, 4, 128
THETA = 10000.0

BM = 256
BQ = 512
BK = 512
SCALE = 1.0 / (D ** 0.5)


def _rope(x, cos, sin):
    rot = jnp.concatenate([-x[:, D // 2:], x[:, : D // 2]], axis=1)
    return x * cos + rot * sin


def _q_proj_kernel(x_ref, wq_ref, cos_ref, sin_ref, q_ref):
    x = x_ref[0]
    cos = cos_ref[...]
    sin = sin_ref[...]
    y = jnp.dot(x, wq_ref[...], preferred_element_type=jnp.float32)
    for h in range(H):
        q_ref[0, h, :, :] = (_rope(y[:, h * D:(h + 1) * D], cos, sin)
                             * SCALE).astype(jnp.bfloat16)


def _kv_proj_kernel(x_ref, wkv_ref, cos_ref, sin_ref, k_ref, v_ref):
    x = x_ref[0]
    cos = cos_ref[...]
    sin = sin_ref[...]
    y = jnp.dot(x, wkv_ref[...], preferred_element_type=jnp.float32)
    for h in range(KVH):
        k_ref[0, h, :, :] = _rope(y[:, h * D:(h + 1) * D], cos, sin).astype(jnp.bfloat16)
        v_ref[0, h, :, :] = y[:, (KVH + h) * D:(KVH + h + 1) * D].astype(jnp.bfloat16)


GROUPS = H // KVH
MQ = GROUPS * BQ


def _flash_kernel(q_ref, k_ref, v_ref, o_ref):
    qi = pl.program_id(2)
    q = q_ref[0].reshape(MQ, D)

    i0 = jax.lax.broadcasted_iota(jnp.int32, (MQ, BK), 0)
    i1 = jax.lax.broadcasted_iota(jnp.int32, (MQ, BK), 1)
    diag_mask = (i0 & (BQ - 1)) >= i1

    m0 = jnp.full((MQ, 1), -1e30, jnp.float32)
    l0 = jnp.zeros((MQ, 1), jnp.float32)
    acc0 = jnp.zeros((MQ, D), jnp.float32)

    def block(kb, carry, masked):
        m, l, acc = carry
        ks = k_ref[0, 0, pl.ds(kb * BK, BK), :]
        vs = v_ref[0, 0, pl.ds(kb * BK, BK), :]
        s = jax.lax.dot_general(q, ks, (((1,), (1,)), ((), ())),
                                preferred_element_type=jnp.float32)
        if masked:
            s = jnp.where(diag_mask, s, -1e30)
        m_new = jnp.maximum(m, s.max(axis=1, keepdims=True))
        alpha = jnp.exp(m - m_new)
        p = jnp.exp(s - m_new)
        l_new = l * alpha + p.sum(axis=1, keepdims=True)
        acc_new = acc * alpha + jnp.dot(p.astype(jnp.bfloat16), vs,
                                        preferred_element_type=jnp.float32)
        return m_new, l_new, acc_new

    carry = jax.lax.fori_loop(0, qi, lambda kb, c: block(kb, c, False),
                              (m0, l0, acc0))
    m, l, acc = block(qi, carry, True)
    o_ref[0] = (acc / l).astype(jnp.bfloat16).reshape(GROUPS, BQ, D)


def _out_proj_kernel(x_ref, wo_ref, o_ref):
    x = jnp.concatenate([x_ref[0, h] for h in range(H)], axis=1)
    o_ref[0] = jnp.dot(x, wo_ref[...], preferred_element_type=jnp.float32)


def kernel(hidden_states, Wq, Wk, Wv, Wo):
    inv_freq = 1.0 / (THETA ** (jnp.arange(0, D, 2, dtype=jnp.float32) / D))
    t = jnp.arange(S, dtype=jnp.float32)
    freqs = jnp.outer(t, inv_freq)
    emb = jnp.concatenate([freqs, freqs], axis=-1)
    cos = jnp.cos(emb)
    sin = jnp.sin(emb)
    wkv = jnp.concatenate([Wk, Wv], axis=1).astype(jnp.bfloat16)
    x16 = hidden_states.astype(jnp.bfloat16)
    wq16 = Wq.astype(jnp.bfloat16)
    wo16 = Wo.astype(jnp.bfloat16)

    q = pl.pallas_call(
        _q_proj_kernel,
        grid=(B, S // BM),
        in_specs=[
            pl.BlockSpec((1, BM, HID), lambda b, m: (b, m, 0)),
            pl.BlockSpec((HID, H * D), lambda b, m: (0, 0)),
            pl.BlockSpec((BM, D), lambda b, m: (m, 0)),
            pl.BlockSpec((BM, D), lambda b, m: (m, 0)),
        ],
        out_specs=pl.BlockSpec((1, H, BM, D), lambda b, m: (b, 0, m, 0)),
        out_shape=jax.ShapeDtypeStruct((B, H, S, D), jnp.bfloat16),
        compiler_params=pltpu.CompilerParams(
            dimension_semantics=("parallel", "arbitrary")),
    )(x16, wq16, cos, sin)

    k, v = pl.pallas_call(
        _kv_proj_kernel,
        grid=(B, S // BM),
        in_specs=[
            pl.BlockSpec((1, BM, HID), lambda b, m: (b, m, 0)),
            pl.BlockSpec((HID, 2 * KVH * D), lambda b, m: (0, 0)),
            pl.BlockSpec((BM, D), lambda b, m: (m, 0)),
            pl.BlockSpec((BM, D), lambda b, m: (m, 0)),
        ],
        out_specs=[
            pl.BlockSpec((1, KVH, BM, D), lambda b, m: (b, 0, m, 0)),
            pl.BlockSpec((1, KVH, BM, D), lambda b, m: (b, 0, m, 0)),
        ],
        out_shape=[
            jax.ShapeDtypeStruct((B, KVH, S, D), jnp.bfloat16),
            jax.ShapeDtypeStruct((B, KVH, S, D), jnp.bfloat16),
        ],
        compiler_params=pltpu.CompilerParams(
            dimension_semantics=("parallel", "arbitrary")),
    )(x16, wkv, cos, sin)

    o = pl.pallas_call(
        _flash_kernel,
        grid=(B, KVH, S // BQ),
        in_specs=[
            pl.BlockSpec((1, GROUPS, BQ, D), lambda b, g, i: (b, g, i, 0)),
            pl.BlockSpec((1, 1, S, D), lambda b, g, i: (b, g, 0, 0)),
            pl.BlockSpec((1, 1, S, D), lambda b, g, i: (b, g, 0, 0)),
        ],
        out_specs=pl.BlockSpec((1, GROUPS, BQ, D), lambda b, g, i: (b, g, i, 0)),
        out_shape=jax.ShapeDtypeStruct((B, H, S, D), jnp.bfloat16),
        compiler_params=pltpu.CompilerParams(
            dimension_semantics=("parallel", "parallel", "arbitrary")),
    )(q, k, v)

    out = pl.pallas_call(
        _out_proj_kernel,
        grid=(B, S // BM),
        in_specs=[
            pl.BlockSpec((1, H, BM, D), lambda b, m: (b, 0, m, 0)),
            pl.BlockSpec((HID, HID), lambda b, m: (0, 0)),
        ],
        out_specs=pl.BlockSpec((1, BM, HID), lambda b, m: (b, m, 0)),
        out_shape=jax.ShapeDtypeStruct((B, S, HID), jnp.float32),
        compiler_params=pltpu.CompilerParams(
            dimension_semantics=("parallel", "arbitrary")),
    )(o, wo16)

    return out


# speedup vs baseline: 2.1332x; 1.0387x over previous
import jax
import jax.numpy as jnp
from jax.experimental import pallas as pl
from jax.experimental.pallas import tpu as pltpu

B, S, HID = 2, 2048, 2048
H, KVH, D = 16, 4, 128
THETA = 10000.0
GROUPS = H // KVH

BM = 256
BQ = 512
BK = 512
MQ = GROUPS * BQ
SCALE = 1.0 / (D ** 0.5)


def _rope(x, cos, sin):
    rot = jnp.concatenate([-x[:, D // 2:], x[:, : D // 2]], axis=1)
    return x * cos + rot * sin


def _qkv_proj_kernel(x_ref, w_ref, cos_ref, sin_ref, q_ref, k_ref, v_ref):
    x = x_ref[0]
    cos = cos_ref[...]
    sin = sin_ref[...]
    y = jnp.dot(x, w_ref[...], preferred_element_type=jnp.float32)
    for h in range(H):
        q_ref[0, h, :, :] = _rope(y[:, h * D:(h + 1) * D], cos, sin).astype(jnp.bfloat16)
    for h in range(KVH):
        c0 = (H + h) * D
        k_ref[0, h, :, :] = _rope(y[:, c0:c0 + D], cos, sin).astype(jnp.bfloat16)
        c1 = (H + KVH + h) * D
        v_ref[0, h, :, :] = y[:, c1:c1 + D].astype(jnp.bfloat16)


def _flash_kernel(q_ref, k_ref, v_ref, wo_ref, o_ref):
    qi = pl.program_id(1)
    g = pl.program_id(2)
    q = q_ref[0].reshape(MQ, D)

    i0 = jax.lax.broadcasted_iota(jnp.int32, (MQ, BK), 0)
    i1 = jax.lax.broadcasted_iota(jnp.int32, (MQ, BK), 1)
    diag_mask = (i0 & (BQ - 1)) >= i1

    m0 = jnp.full((MQ, 1), -1e30, jnp.float32)
    l0 = jnp.zeros((MQ, 1), jnp.float32)
    acc0 = jnp.zeros((MQ, D), jnp.float32)

    def block(kb, carry, masked):
        m, l, acc = carry
        ks = k_ref[0, 0, pl.ds(kb * BK, BK), :]
        vs = v_ref[0, 0, pl.ds(kb * BK, BK), :]
        s = jax.lax.dot_general(q, ks, (((1,), (1,)), ((), ())),
                                preferred_element_type=jnp.float32)
        if masked:
            s = jnp.where(diag_mask, s, -1e30)
        m_new = jnp.maximum(m, s.max(axis=1, keepdims=True))
        alpha = jnp.exp(m - m_new)
        p = jnp.exp(s - m_new)
        l_new = l * alpha + p.sum(axis=1, keepdims=True)
        acc_new = acc * alpha + jnp.dot(p.astype(jnp.bfloat16), vs,
                                        preferred_element_type=jnp.float32)
        return m_new, l_new, acc_new

    carry = jax.lax.fori_loop(0, qi, lambda kb, c: block(kb, c, False),
                              (m0, l0, acc0))
    m, l, acc = block(qi, carry, True)

    attn = (acc / l).astype(jnp.bfloat16)
    attn_w = jnp.concatenate(
        [attn[j * BQ:(j + 1) * BQ, :] for j in range(GROUPS)], axis=1)
    contrib = jnp.dot(attn_w, wo_ref[...], preferred_element_type=jnp.float32)

    @pl.when(g == 0)
    def _():
        o_ref[0] = contrib

    @pl.when(g != 0)
    def _():
        o_ref[0] += contrib


def kernel(hidden_states, Wq, Wk, Wv, Wo):
    inv_freq = 1.0 / (THETA ** (jnp.arange(0, D, 2, dtype=jnp.float32) / D))
    t = jnp.arange(S, dtype=jnp.float32)
    freqs = jnp.outer(t, inv_freq)
    emb = jnp.concatenate([freqs, freqs], axis=-1)
    cos = jnp.cos(emb)
    sin = jnp.sin(emb)
    wqkv = jnp.concatenate([Wq * SCALE, Wk, Wv], axis=1).astype(jnp.bfloat16)
    wo16 = Wo.astype(jnp.bfloat16)
    x16 = hidden_states.astype(jnp.bfloat16)

    q, k, v = pl.pallas_call(
        _qkv_proj_kernel,
        grid=(B, S // BM),
        in_specs=[
            pl.BlockSpec((1, BM, HID), lambda b, m: (b, m, 0)),
            pl.BlockSpec((HID, (H + 2 * KVH) * D), lambda b, m: (0, 0)),
            pl.BlockSpec((BM, D), lambda b, m: (m, 0)),
            pl.BlockSpec((BM, D), lambda b, m: (m, 0)),
        ],
        out_specs=[
            pl.BlockSpec((1, H, BM, D), lambda b, m: (b, 0, m, 0)),
            pl.BlockSpec((1, KVH, BM, D), lambda b, m: (b, 0, m, 0)),
            pl.BlockSpec((1, KVH, BM, D), lambda b, m: (b, 0, m, 0)),
        ],
        out_shape=[
            jax.ShapeDtypeStruct((B, H, S, D), jnp.bfloat16),
            jax.ShapeDtypeStruct((B, KVH, S, D), jnp.bfloat16),
            jax.ShapeDtypeStruct((B, KVH, S, D), jnp.bfloat16),
        ],
        compiler_params=pltpu.CompilerParams(
            dimension_semantics=("parallel", "arbitrary")),
    )(x16, wqkv, cos, sin)

    out = pl.pallas_call(
        _flash_kernel,
        grid=(B, S // BQ, KVH),
        in_specs=[
            pl.BlockSpec((1, GROUPS, BQ, D), lambda b, i, g: (b, g, i, 0)),
            pl.BlockSpec((1, 1, S, D), lambda b, i, g: (b, g, 0, 0)),
            pl.BlockSpec((1, 1, S, D), lambda b, i, g: (b, g, 0, 0)),
            pl.BlockSpec((GROUPS * D, HID), lambda b, i, g: (g, 0)),
        ],
        out_specs=pl.BlockSpec((1, BQ, HID), lambda b, i, g: (b, i, 0)),
        out_shape=jax.ShapeDtypeStruct((B, S, HID), jnp.float32),
        compiler_params=pltpu.CompilerParams(
            dimension_semantics=("parallel", "arbitrary", "arbitrary")),
    )(q, k, v, wo16)

    return out
